# baseline (speedup 1.0000x reference)
import jax
import jax.numpy as jnp
from jax import lax
from jax.experimental import pallas as pl
from jax.experimental.pallas import tpu as pltpu
from jax.experimental.pallas import tpu_sc as plsc

NC = 2
NS = 16
NW = NC * NS

BATCH_N = 4096
SEQ_LEN = 200
B = BATCH_N * SEQ_LEN
D = 64
BPW = B // NW
CHUNK = 128
NCHUNK = BPW // CHUNK
TB = BATCH_N // 128


def _gather_body(idx_hbm, table_hbm, out_hbm, idx_v, rows_v, tiles_v, sems, wsems):
    wid = lax.axis_index("s") * NC + lax.axis_index("c")
    base = wid * BPW
    pltpu.sync_copy(idx_hbm.at[pl.ds(base, BPW)], idx_v)

    lanes = [lax.iota(jnp.int32, 16) + 16 * g for g in range(4)]

    def gather(c, b):
        pltpu.async_copy(
            table_hbm.at[idx_v.at[pl.ds(c * CHUNK, CHUNK)]],
            rows_v.at[b],
            sems.at[b],
        )

    gather(0, 0)

    def body(c, carry):
        b = lax.rem(c, 2)
        pltpu.make_async_copy(
            table_hbm.at[idx_v.at[pl.ds(0, CHUNK)]], rows_v.at[b], sems.at[b]
        ).wait()

        @pl.when(c + 1 < NCHUNK)
        def _():
            gather(c + 1, 1 - b)

        @pl.when(c >= 2)
        def _():
            for f8 in range(8):
                pltpu.make_async_copy(
                    tiles_v.at[b, pl.ds(f8 * 8, 8), pl.ds(0, 128)],
                    out_hbm.at[0, f8, 0],
                    wsems.at[b],
                ).wait()

        rb = rows_v.at[b]
        b_vec = jnp.full((16,), b, jnp.int32)
        for bj in range(CHUNK):
            bj_vec = jnp.full((16,), bj, jnp.int32)
            for k in range(4):
                val = rb[bj, pl.ds(k * 16, 16)]
                plsc.store_scatter(tiles_v, [b_vec, lanes[k], bj_vec], val)

        p0 = base + c * CHUNK
        s = p0 // BATCH_N
        b32 = (p0 - s * BATCH_N) // 128
        for f8 in range(8):
            pltpu.async_copy(
                tiles_v.at[b, pl.ds(f8 * 8, 8), pl.ds(0, 128)],
                out_hbm.at[s, f8, b32],
                wsems.at[b],
            )
        return carry

    lax.fori_loop(0, NCHUNK, body, 0)

    for b in range(2):
        for f8 in range(8):
            pltpu.make_async_copy(
                tiles_v.at[b, pl.ds(f8 * 8, 8), pl.ds(0, 128)],
                out_hbm.at[0, f8, 0],
                wsems.at[b],
            ).wait()


def kernel(src_seq, src_mask, emb_table):
    del src_mask
    mesh = plsc.VectorSubcoreMesh(core_axis_name="c", subcore_axis_name="s")

    idx = jnp.transpose(src_seq).reshape(B)

    out5 = pl.kernel(
        _gather_body,
        out_type=jax.ShapeDtypeStruct((SEQ_LEN, 8, TB, 8, 128), jnp.float32),
        mesh=mesh,
        compiler_params=pltpu.CompilerParams(use_tc_tiling_on_sc=False, needs_layout_passes=False),
        scratch_types=[
            pltpu.VMEM((BPW,), jnp.int32),
            pltpu.VMEM((2, CHUNK, D), jnp.float32),
            pltpu.VMEM((2, 64, 129), jnp.float32),
            pltpu.SemaphoreType.DMA((2,)),
            pltpu.SemaphoreType.DMA((2,)),
        ],
    )(idx, emb_table)
    return out5.transpose(2, 4, 0, 1, 3).reshape(BATCH_N, SEQ_LEN, D)


# speedup vs baseline: 1.0036x; 1.0036x over previous
import jax
import jax.numpy as jnp
from jax import lax
from jax.experimental import pallas as pl
from jax.experimental.pallas import tpu as pltpu
from jax.experimental.pallas import tpu_sc as plsc

NC = 2
NS = 16
NW = NC * NS

BATCH_N = 4096
SEQ_LEN = 200
B = BATCH_N * SEQ_LEN
D = 64
BPW = B // NW
CHUNK = 128
NCHUNK = BPW // CHUNK
TB = BATCH_N // 128


def _gather_body(idx_hbm, table_hbm, out_hbm, idx_v, rows_v, tiles_v, sems, wsems):
    wid = lax.axis_index("s") * NC + lax.axis_index("c")
    base = wid * BPW
    pltpu.sync_copy(idx_hbm.at[pl.ds(base, BPW)], idx_v)

    lanes = [lax.iota(jnp.int32, 16) + 16 * g for g in range(4)]

    def gather(c, b):
        pltpu.async_copy(
            table_hbm.at[idx_v.at[pl.ds(c * CHUNK, CHUNK)]],
            rows_v.at[b],
            sems.at[b],
        )

    gather(0, 0)

    def body(c, carry):
        b = lax.rem(c, 2)
        pltpu.make_async_copy(
            table_hbm.at[idx_v.at[pl.ds(0, CHUNK)]], rows_v.at[b], sems.at[b]
        ).wait()

        @pl.when(c + 1 < NCHUNK)
        def _():
            gather(c + 1, 1 - b)

        @pl.when(c >= 2)
        def _():
            for f8 in range(8):
                pltpu.make_async_copy(
                    tiles_v.at[b, pl.ds(f8 * 8, 8), pl.ds(0, 128)],
                    out_hbm.at[0, f8, 0],
                    wsems.at[b],
                ).wait()

        rb = rows_v.at[b]
        tb = tiles_v.at[b]
        for bj in range(CHUNK):
            bj_vec = jnp.full((16,), bj, jnp.int32)
            for k in range(4):
                val = rb[bj, pl.ds(k * 16, 16)]
                plsc.store_scatter(tb, [lanes[k], bj_vec], val)

        p0 = base + c * CHUNK
        s = p0 // BATCH_N
        b32 = (p0 - s * BATCH_N) // 128
        for f8 in range(8):
            pltpu.async_copy(
                tiles_v.at[b, pl.ds(f8 * 8, 8), pl.ds(0, 128)],
                out_hbm.at[s, f8, b32],
                wsems.at[b],
            )
        return carry

    lax.fori_loop(0, NCHUNK, body, 0)

    for b in range(2):
        for f8 in range(8):
            pltpu.make_async_copy(
                tiles_v.at[b, pl.ds(f8 * 8, 8), pl.ds(0, 128)],
                out_hbm.at[0, f8, 0],
                wsems.at[b],
            ).wait()


def kernel(src_seq, src_mask, emb_table):
    del src_mask
    mesh = plsc.VectorSubcoreMesh(core_axis_name="c", subcore_axis_name="s")

    idx = jnp.transpose(src_seq).reshape(B)

    out5 = pl.kernel(
        _gather_body,
        out_type=jax.ShapeDtypeStruct((SEQ_LEN, 8, TB, 8, 128), jnp.float32),
        mesh=mesh,
        compiler_params=pltpu.CompilerParams(use_tc_tiling_on_sc=False, needs_layout_passes=False),
        scratch_types=[
            pltpu.VMEM((BPW,), jnp.int32),
            pltpu.VMEM((2, CHUNK, D), jnp.float32),
            pltpu.VMEM((2, 64, 129), jnp.float32),
            pltpu.SemaphoreType.DMA((2,)),
            pltpu.SemaphoreType.DMA((2,)),
        ],
    )(idx, emb_table)
    return out5.transpose(2, 4, 0, 1, 3).reshape(BATCH_N, SEQ_LEN, D)
